# Initial kernel scaffold; baseline (speedup 1.0000x reference)
import jax
import jax.numpy as jnp
from jax.experimental import pallas as pl


def kernel(i):
    raise NotImplementedError("write your pallas kernel here")


# speedup vs baseline: 6.6431x; 6.6431x over previous
import functools

import jax
import jax.numpy as jnp
from jax import lax
from jax.experimental import pallas as pl
from jax.experimental.pallas import tpu as pltpu
from jax.experimental.pallas import tpu_sc as plsc

ROWS = 128
N = 32768
TOPK = 512
NV = N // 16
NC = 2
NS = 16
ROWS_PER_W = ROWS // (NC * NS)
MASK15 = 32767


def _vsort(x, descending):
  ks, _ = plsc.sort_key_val(x, x, descending=descending)
  return ks


def _bitonic_merge(xs, descending):
  n = len(xs)
  if n == 1:
    return [_vsort(xs[0], descending)]
  half = n // 2
  lo, hi = [], []
  for i in range(half):
    a, b = xs[i], xs[i + half]
    mx, mn = jnp.maximum(a, b), jnp.minimum(a, b)
    if descending:
      lo.append(mx)
      hi.append(mn)
    else:
      lo.append(mn)
      hi.append(mx)
  return _bitonic_merge(lo, descending) + _bitonic_merge(hi, descending)


def _sort512(ref, descending):
  def base_body(j, _):
    ref[pl.ds(j * 16, 16)] = _vsort(ref[pl.ds(j * 16, 16)], descending)
    return 0

  lax.fori_loop(0, 32, base_body, 0, unroll=False)

  for h in (1, 2, 4, 8, 16):
    def merge_body(mi, _, h=h):
      base = mi * (2 * h * 16)
      a = [ref[pl.ds(base + k * 16, 16)] for k in range(h)]
      b = [ref[pl.ds(base + (h + k) * 16, 16)] for k in range(h)]
      xs = a + [lax.rev(v, (0,)) for v in reversed(b)]
      xs = _bitonic_merge(xs, descending)
      for k in range(2 * h):
        ref[pl.ds(base + k * 16, 16)] = xs[k]
      return 0

    lax.fori_loop(0, 16 // h, merge_body, 0, unroll=False)


def _body(i_hbm, ids_hbm, vals_hbm,
          row_a, row_b, cidx, hist, totals, ck, ci, skey, comp,
          stage_ids, stage_vals, scr_a, scr_b, sem_a, sem_b):
  lane = lax.iota(jnp.int32, 16)
  zeros16 = jnp.zeros((16,), jnp.int32)
  ones16 = jnp.ones((16,), jnp.int32)
  full512 = jnp.full((16,), TOPK, jnp.int32)
  full15 = jnp.full((16,), 15, jnp.int32)

  wid = lax.axis_index("s") * NC + lax.axis_index("c")
  r0 = wid * ROWS_PER_W

  def to_key(v):
    b = plsc.bitcast(v, jnp.int32)
    return b ^ ((b >> 31) & 0x7FFFFFFF)

  def find_digit(target):
    def tphase(t, _):
      acc = zeros16
      for l in range(16):
        acc = acc + hist[pl.ds(l * 256 + t * 16, 16)]
        hist[pl.ds(l * 256 + t * 16, 16)] = zeros16
      totals[pl.ds(t * 16, 16)] = acc
      return 0

    lax.fori_loop(0, 16, tphase, 0, unroll=False)

    def sphase(t2, carry):
      found, dig, cab, csum = carry
      t = 15 - t2
      v = totals[pl.ds(t * 16, 16)]
      rv = lax.rev(v, (0,))
      cs = plsc.cumsum(rv)
      gcs = cs + csum
      mask = (gcs >= target) & (found == 0)
      p = plsc.all_reduce_ffs(mask)
      hit = (p < 16) & (found == 0)
      pc = jnp.minimum(p, full15)
      scr_a[...] = gcs
      scr_b[...] = rv
      gcs_p = plsc.load_gather(scr_a, [pc])
      rv_p = plsc.load_gather(scr_b, [pc])
      dig_if = t * 16 + 15 - pc
      cab_if = gcs_p - rv_p
      found2 = jnp.where(hit, ones16, found)
      dig2 = jnp.where(hit, dig_if, dig)
      cab2 = jnp.where(hit, cab_if, cab)
      csum2 = plsc.load_gather(scr_a, [full15])
      return found2, dig2, cab2, csum2

    _, dig, cab, _ = lax.fori_loop(
        0, 16, sphase, (zeros16, zeros16, zeros16, zeros16), unroll=False)
    return dig, cab

  def process_row(vrow, row):
    def p1(j, _):
      key = to_key(vrow[pl.ds(j * 16, 16)])
      d1 = (key >> 24) + 128
      plsc.addupdate_scatter(hist, [d1 + lane * 256], ones16)
      return 0

    lax.fori_loop(0, NV, p1, 0, unroll=False)

    b1, ca1 = find_digit(full512)
    t2 = full512 - ca1
    p1s = b1 - 128

    def p2(j, cnt):
      key = to_key(vrow[pl.ds(j * 16, 16)])
      d1 = (key >> 24) + 128
      mge = d1 >= b1
      plsc.store_compressed(cidx.at[pl.ds(cnt, 16)], j * 16 + lane, mask=mge)
      return cnt + jnp.max(plsc.all_reduce_population_count(mge))

    c_cnt = lax.fori_loop(0, NV, p2, jnp.int32(0), unroll=False)
    ncv = (c_cnt + 15) >> 4
    c_cnt_s = jnp.full((16,), c_cnt, jnp.int32)

    def ckeys(j):
      idx = cidx[pl.ds(j * 16, 16)]
      lm = (j * 16 + lane) < c_cnt_s
      v = plsc.load_gather(vrow, [idx], mask=lm)
      return to_key(v), idx, lm

    def refine(prefix, shift, digit_shift, target):
      def pr(j, _):
        key, _, lm = ckeys(j)
        mk = ((key >> shift) == prefix) & lm
        d = (key >> digit_shift) & 0xFF
        plsc.addupdate_scatter(hist, [d + lane * 256], ones16, mask=mk)
        return 0

      lax.fori_loop(0, ncv, pr, 0, unroll=False)
      dig, cab = find_digit(target)
      return (prefix << 8) | dig, target - cab

    p2s, t3 = refine(p1s, 24, 16, t2)
    p3s, t4 = refine(p2s, 16, 8, t3)
    thr, t5 = refine(p3s, 8, 0, t4)
    m = t5

    def p6(j, carry):
      cnt, eqc = carry
      key, idx, lm = ckeys(j)
      gt = (key > thr) & lm
      eq = (key == thr) & lm
      incl = plsc.cumsum(eq.astype(jnp.int32))
      take = gt | (eq & ((eqc + incl - 1) < m))
      plsc.store_compressed(ck.at[pl.ds(cnt, 16)], key, mask=take)
      plsc.store_compressed(ci.at[pl.ds(cnt, 16)], idx, mask=take)
      cnt = cnt + jnp.max(plsc.all_reduce_population_count(take))
      eqc = eqc + plsc.all_reduce_population_count(eq)
      return cnt, eqc

    lax.fori_loop(0, ncv, p6, (jnp.int32(0), zeros16), unroll=False)

    def cp(j, _):
      skey[pl.ds(j * 16, 16)] = ck[pl.ds(j * 16, 16)]
      return 0

    lax.fori_loop(0, 32, cp, 0, unroll=False)
    _sort512(skey, descending=True)

    def bs(j, _):
      key = ck[pl.ds(j * 16, 16)]
      idx = ci[pl.ds(j * 16, 16)]
      lo = zeros16
      for step in (256, 128, 64, 32, 16, 8, 4, 2, 1):
        t = lo + step
        g = plsc.load_gather(skey, [t - 1])
        lo = jnp.where(g > key, t, lo)
      comp[pl.ds(j * 16, 16)] = (lo << 15) | idx
      return 0

    lax.fori_loop(0, 32, bs, 0, unroll=False)
    _sort512(comp, descending=False)

    def emit(j, _):
      c = comp[pl.ds(j * 16, 16)]
      stage_ids[pl.ds(j * 16, 16)] = (c & MASK15).astype(jnp.float32)
      s = skey[pl.ds(j * 16, 16)]
      b = s ^ ((s >> 31) & 0x7FFFFFFF)
      stage_vals[pl.ds(j * 16, 16)] = plsc.bitcast(b, jnp.float32)
      return 0

    lax.fori_loop(0, 32, emit, 0, unroll=False)
    pltpu.sync_copy(stage_ids, ids_hbm.at[row])
    pltpu.sync_copy(stage_vals, vals_hbm.at[row])

  def z(j, _):
    hist[pl.ds(j * 16, 16)] = zeros16
    return 0

  lax.fori_loop(0, 256, z, 0, unroll=False)

  bufs = (row_a, row_b)
  sems = (sem_a, sem_b)
  pltpu.make_async_copy(i_hbm.at[r0], row_a, sem_a).start()
  for rr in range(ROWS_PER_W):
    cur, csem = bufs[rr % 2], sems[rr % 2]
    pltpu.make_async_copy(i_hbm.at[r0 + rr], cur, csem).wait()
    if rr + 1 < ROWS_PER_W:
      nxt, nsem = bufs[(rr + 1) % 2], sems[(rr + 1) % 2]
      pltpu.make_async_copy(i_hbm.at[r0 + rr + 1], nxt, nsem).start()
    process_row(cur, r0 + rr)


@functools.partial(jax.jit, static_argnums=())
def kernel(i):
  mesh = plsc.VectorSubcoreMesh(
      core_axis_name="c", subcore_axis_name="s",
      num_cores=NC, num_subcores=NS)
  call = pl.kernel(
      _body,
      out_type=(
          jax.ShapeDtypeStruct((ROWS, TOPK), jnp.float32),
          jax.ShapeDtypeStruct((ROWS, TOPK), jnp.float32),
      ),
      mesh=mesh,
      compiler_params=pltpu.CompilerParams(needs_layout_passes=False),
      scratch_types=[
          pltpu.VMEM((N,), jnp.float32),
          pltpu.VMEM((N,), jnp.float32),
          pltpu.VMEM((N,), jnp.int32),
          pltpu.VMEM((4096,), jnp.int32),
          pltpu.VMEM((256,), jnp.int32),
          pltpu.VMEM((TOPK + 32,), jnp.int32),
          pltpu.VMEM((TOPK + 32,), jnp.int32),
          pltpu.VMEM((TOPK,), jnp.int32),
          pltpu.VMEM((TOPK,), jnp.int32),
          pltpu.VMEM((TOPK,), jnp.float32),
          pltpu.VMEM((TOPK,), jnp.float32),
          pltpu.VMEM((16,), jnp.int32),
          pltpu.VMEM((16,), jnp.int32),
          pltpu.SemaphoreType.DMA,
          pltpu.SemaphoreType.DMA,
      ],
  )
  ids, vals = call(i)
  return ids, vals
